# baseline (speedup 1.0000x reference)
import functools

import jax
import jax.numpy as jnp
from jax import lax
from jax.experimental import pallas as pl
from jax.experimental.pallas import tpu as pltpu
from jax.experimental.pallas import tpu_sc as plsc


def _make_gather(V, D, DP, NJ, NI):
  info = plsc.get_sparse_core_info()
  NC, NS, L = info.num_cores, info.num_subcores, info.num_lanes
  NW = NC * NS
  B = NJ * NI
  assert B % NW == 0
  b_per_w = B // NW
  CHUNK = 256
  NB = 2
  assert b_per_w % (NB * CHUNK) == 0
  assert NI % CHUNK == 0 and b_per_w % CHUNK == 0
  n_chunks = b_per_w // CHUNK
  QK = D // L

  mesh = plsc.VectorSubcoreMesh(core_axis_name="c", subcore_axis_name="s")

  @functools.partial(
      pl.kernel,
      mesh=mesh,
      compiler_params=pltpu.CompilerParams(needs_layout_passes=False),
      out_type=jax.ShapeDtypeStruct((NJ, D, NI), jnp.float32),
      scratch_types=[
          pltpu.VMEM((b_per_w,), jnp.int32),
          [pltpu.VMEM((CHUNK, DP), jnp.float32) for _ in range(NB)],
          pltpu.VMEM((D, CHUNK), jnp.float32),
          [pltpu.SemaphoreType.DMA for _ in range(NB)],
      ],
  )
  def k(x_hbm, table_hbm, out_hbm, idx_v, rows, trans, sems):
    wid = lax.axis_index("s") * NC + lax.axis_index("c")
    base = wid * b_per_w
    pltpu.sync_copy(x_hbm.at[pl.ds(base, b_per_w)], idx_v)

    iotas = [lax.iota(jnp.int32, L) + q * L for q in range(QK)]

    def idx_slice(c):
      return idx_v.at[pl.ds(pl.multiple_of(c * CHUNK, CHUNK), CHUNK)]

    def start_gather(c, buf):
      pltpu.async_copy(table_hbm.at[idx_slice(c)], rows[buf], sems[buf])

    for b in range(NB):
      start_gather(b, b)

    def transpose_chunk(buf):
      UNROLL = 4

      def tbody(it, carry):
        for u in range(UNROLL):
          r = it * UNROLL + u
          rcol = jnp.full((L,), 0, jnp.int32) + r
          for q in range(QK):
            v = plsc.load_gather(rows[buf], [rcol, iotas[q]])
            plsc.store_scatter(trans, [iotas[q], rcol], v)
        return carry

      lax.fori_loop(0, CHUNK // UNROLL, tbody, 0)

    def step(c, buf):
      pltpu.make_async_copy(table_hbm.at[idx_slice(c)], rows[buf],
                            sems[buf]).wait()
      transpose_chunk(buf)

      @pl.when(c + NB < n_chunks)
      def _():
        start_gather(c + NB, buf)

      n0 = base + c * CHUNK
      j = n0 // NI
      i0 = n0 % NI
      for h in range(CHUNK // 128):
        pltpu.sync_copy(
            trans.at[:, pl.ds(h * 128, 128)],
            out_hbm.at[j, :, pl.ds(pl.multiple_of(i0 + h * 128, 128), 128)])

    def loop_body(i, carry):
      for b in range(NB):
        step(NB * i + b, b)
      return carry

    lax.fori_loop(0, n_chunks // NB, loop_body, 0)

  return k


_gather_cache = {}


def kernel(x, weight):
  V, D = weight.shape
  DP = 128
  n_i, n_j = x.shape
  key = (V, D, DP, n_j, n_i)
  if key not in _gather_cache:
    _gather_cache[key] = _make_gather(V, D, DP, n_j, n_i)
  xt = jnp.transpose(x).reshape(-1).astype(jnp.int32)
  wp = jnp.concatenate(
      [weight, jnp.zeros((V, DP - D), jnp.float32)], axis=1)
  out = _gather_cache[key](xt, wp)
  return jnp.transpose(out, (2, 0, 1))


# speedup vs baseline: 1.0024x; 1.0024x over previous
import functools

import jax
import jax.numpy as jnp
from jax import lax
from jax.experimental import pallas as pl
from jax.experimental.pallas import tpu as pltpu
from jax.experimental.pallas import tpu_sc as plsc


def _make_gather(V, D, DP, NJ, NI):
  info = plsc.get_sparse_core_info()
  NC, NS, L = info.num_cores, info.num_subcores, info.num_lanes
  NW = NC * NS
  B = NJ * NI
  assert B % NW == 0
  b_per_w = B // NW
  CHUNK = 256
  NB = 2
  assert b_per_w % (NB * CHUNK) == 0
  assert NI % CHUNK == 0 and b_per_w % CHUNK == 0
  n_chunks = b_per_w // CHUNK
  QK = D // L

  mesh = plsc.VectorSubcoreMesh(core_axis_name="c", subcore_axis_name="s")

  @functools.partial(
      pl.kernel,
      mesh=mesh,
      compiler_params=pltpu.CompilerParams(needs_layout_passes=False),
      out_type=jax.ShapeDtypeStruct((NJ, D, NI), jnp.float32),
      scratch_types=[
          pltpu.VMEM((b_per_w,), jnp.int32),
          [pltpu.VMEM((CHUNK, DP), jnp.float32) for _ in range(NB)],
          pltpu.VMEM((D, CHUNK + 1), jnp.float32),
          [pltpu.SemaphoreType.DMA for _ in range(NB)],
      ],
  )
  def k(x_hbm, table_hbm, out_hbm, idx_v, rows, trans, sems):
    wid = lax.axis_index("s") * NC + lax.axis_index("c")
    base = wid * b_per_w
    pltpu.sync_copy(x_hbm.at[pl.ds(base, b_per_w)], idx_v)

    iotas = [lax.iota(jnp.int32, L) + q * L for q in range(QK)]

    def idx_slice(c):
      return idx_v.at[pl.ds(pl.multiple_of(c * CHUNK, CHUNK), CHUNK)]

    def start_gather(c, buf):
      pltpu.async_copy(table_hbm.at[idx_slice(c)], rows[buf], sems[buf])

    for b in range(NB):
      start_gather(b, b)

    def transpose_chunk(buf):
      UNROLL = 4

      def tbody(it, carry):
        for u in range(UNROLL):
          r = it * UNROLL + u
          rcol = jnp.full((L,), 0, jnp.int32) + r
          for q in range(QK):
            v = plsc.load_gather(rows[buf], [rcol, iotas[q]])
            plsc.store_scatter(trans, [iotas[q], rcol], v)
        return carry

      lax.fori_loop(0, CHUNK // UNROLL, tbody, 0)

    def step(c, buf):
      pltpu.make_async_copy(table_hbm.at[idx_slice(c)], rows[buf],
                            sems[buf]).wait()
      transpose_chunk(buf)

      @pl.when(c + NB < n_chunks)
      def _():
        start_gather(c + NB, buf)

      n0 = base + c * CHUNK
      j = n0 // NI
      i0 = n0 % NI
      for h in range(CHUNK // 128):
        pltpu.sync_copy(
            trans.at[:, pl.ds(h * 128, 128)],
            out_hbm.at[j, :, pl.ds(pl.multiple_of(i0 + h * 128, 128), 128)])

    def loop_body(i, carry):
      for b in range(NB):
        step(NB * i + b, b)
      return carry

    lax.fori_loop(0, n_chunks // NB, loop_body, 0)

  return k


_gather_cache = {}


def kernel(x, weight):
  V, D = weight.shape
  DP = 128
  n_i, n_j = x.shape
  key = (V, D, DP, n_j, n_i)
  if key not in _gather_cache:
    _gather_cache[key] = _make_gather(V, D, DP, n_j, n_i)
  xt = jnp.transpose(x).reshape(-1).astype(jnp.int32)
  wp = jnp.concatenate(
      [weight, jnp.zeros((V, DP - D), jnp.float32)], axis=1)
  out = _gather_cache[key](xt, wp)
  return jnp.transpose(out, (2, 0, 1))


# speedup vs baseline: 1.4027x; 1.3994x over previous
import functools

import jax
import jax.numpy as jnp
from jax import lax
from jax.experimental import pallas as pl
from jax.experimental.pallas import tpu as pltpu
from jax.experimental.pallas import tpu_sc as plsc


def _make_gather(V2, D, B):
  info = plsc.get_sparse_core_info()
  NC, NS, L = info.num_cores, info.num_subcores, info.num_lanes
  NW = NC * NS
  assert B % NW == 0
  b_per_w = B // NW
  CHUNK = 256
  NB = 2
  assert b_per_w % (NB * CHUNK) == 0
  n_chunks = b_per_w // CHUNK

  mesh = plsc.VectorSubcoreMesh(core_axis_name="c", subcore_axis_name="s")

  @functools.partial(
      pl.kernel,
      mesh=mesh,
      compiler_params=pltpu.CompilerParams(
          use_tc_tiling_on_sc=False, disable_bounds_checks=True),
      out_type=jax.ShapeDtypeStruct((B, D), jnp.float32),
      scratch_types=[
          pltpu.VMEM((b_per_w,), jnp.int32),
          [pltpu.VMEM((CHUNK, D), jnp.float32) for _ in range(NB)],
          [pltpu.SemaphoreType.DMA for _ in range(NB)],
      ],
  )
  def k(x_hbm, table_hbm, out_hbm, idx_v, rows, sems):
    wid = lax.axis_index("s") * NC + lax.axis_index("c")
    base = wid * b_per_w
    pltpu.sync_copy(x_hbm.at[pl.ds(base, b_per_w)], idx_v)

    def dbl(i, carry):
      sl = idx_v.at[pl.ds(pl.multiple_of(i * L, L), L)]
      sl[...] = sl[...] * 2
      return carry

    lax.fori_loop(0, b_per_w // L, dbl, 0)

    def idx_slice(c):
      return idx_v.at[pl.ds(pl.multiple_of(c * CHUNK, CHUNK), CHUNK)]

    def start_gather(c, buf):
      pltpu.async_copy(table_hbm.at[idx_slice(c)], rows[buf], sems[buf])

    for b in range(NB):
      start_gather(b, b)

    def step(c, buf):
      pltpu.make_async_copy(table_hbm.at[idx_slice(c)], rows[buf],
                            sems[buf]).wait()
      pltpu.sync_copy(
          rows[buf],
          out_hbm.at[pl.ds(pl.multiple_of(base + c * CHUNK, CHUNK), CHUNK)])

      @pl.when(c + NB < n_chunks)
      def _():
        start_gather(c + NB, buf)

    def loop_body(i, carry):
      for b in range(NB):
        step(NB * i + b, b)
      return carry

    lax.fori_loop(0, n_chunks // NB, loop_body, 0)

  return k


_gather_cache = {}


def kernel(x, weight):
  V, D = weight.shape
  DP = 128
  n_i, n_j = x.shape
  B = n_i * n_j
  key = (2 * V, D, B)
  if key not in _gather_cache:
    _gather_cache[key] = _make_gather(2 * V, D, B)
  xt = jnp.transpose(x).reshape(-1).astype(jnp.int32)
  wp = jnp.concatenate(
      [weight, jnp.zeros((V, DP - D), jnp.float32)], axis=1)
  w2 = wp.reshape(2 * V, D)
  out = _gather_cache[key](xt, w2)
  return jnp.transpose(out.reshape(n_j, n_i, D), (1, 0, 2))
